# Initial kernel scaffold; baseline (speedup 1.0000x reference)
import jax
import jax.numpy as jnp
from jax.experimental import pallas as pl


def kernel(k_tensor, token_indices, emb, P):
    raise NotImplementedError("write your pallas kernel here")


# speedup vs baseline: 1.0386x; 1.0386x over previous
import functools

import numpy as np
import jax
import jax.numpy as jnp
from jax import lax
from jax.experimental import pallas as pl
from jax.experimental.pallas import tpu as pltpu
from jax.experimental.pallas import tpu_sc as plsc

M = 64
Q = M * M
B_TOTAL = 16384
BB = 512

NC = 2
NS = 16
NW = NC * NS
ROWS_PER_W = B_TOTAL // NW
IDX_CHUNK = 128
CHUNKS = ROWS_PER_W // IDX_CHUNK

_W_CONST = (2.0 * np.pi / (np.arange(Q, dtype=np.float64) + 2.0)).astype(
    np.float32).reshape(1, Q)
_D_CONST = np.tile(np.eye(M, dtype=np.float32), (1, M))
_E_CONST = np.repeat(np.eye(M, dtype=np.float32), M, axis=0)


def _sc_gather(tok2d, emb):
    mesh = plsc.VectorSubcoreMesh(core_axis_name="c", subcore_axis_name="s")

    @functools.partial(
        pl.kernel,
        mesh=mesh,
        out_type=jax.ShapeDtypeStruct((B_TOTAL, M), jnp.float32),
        scratch_types=[
            pltpu.VMEM((CHUNKS, IDX_CHUNK), jnp.int32),
            pltpu.VMEM((ROWS_PER_W, M), jnp.float32),
            pltpu.SemaphoreType.DMA,
        ],
        compiler_params=pltpu.CompilerParams(use_tc_tiling_on_sc=False),
    )
    def gk(tok_hbm, emb_hbm, out_hbm, idx_v, rows_v, sem):
        wid = lax.axis_index("s") * NC + lax.axis_index("c")
        pltpu.sync_copy(tok_hbm.at[pl.ds(wid * CHUNKS, CHUNKS)], idx_v)
        copies = [
            pltpu.async_copy(
                emb_hbm.at[idx_v.at[j]],
                rows_v.at[pl.ds(j * IDX_CHUNK, IDX_CHUNK)],
                sem,
            )
            for j in range(CHUNKS)
        ]
        for c in copies:
            c.wait()
        pltpu.sync_copy(rows_v, out_hbm.at[pl.ds(wid * ROWS_PER_W, ROWS_PER_W)])

    return gk(tok2d, emb)


def _dd_body(k_ref, x_ref, d2_ref, e_ref, w_ref, o_ref):
    ang = k_ref[...] * w_ref[...]
    g = jnp.cos(ang)
    xt = jnp.dot(x_ref[...], d2_ref[...],
                 preferred_element_type=jnp.float32)
    o_ref[...] = jnp.dot(g * xt, e_ref[...],
                         preferred_element_type=jnp.float32)


def _dense_tc(k2d, x, d2):
    grid = (k2d.shape[0] // BB,)
    return pl.pallas_call(
        _dd_body,
        grid=grid,
        in_specs=[
            pl.BlockSpec((BB, 1), lambda i: (i, 0)),
            pl.BlockSpec((BB, M), lambda i: (i, 0)),
            pl.BlockSpec((M, Q), lambda i: (0, 0)),
            pl.BlockSpec((Q, M), lambda i: (0, 0)),
            pl.BlockSpec((1, Q), lambda i: (0, 0)),
        ],
        out_specs=pl.BlockSpec((BB, M), lambda i: (i, 0)),
        out_shape=jax.ShapeDtypeStruct((k2d.shape[0], M), jnp.float32),
        compiler_params=pltpu.CompilerParams(
            dimension_semantics=("parallel",)),
    )(k2d, x, d2, jnp.asarray(_E_CONST), jnp.asarray(_W_CONST))


def kernel(k_tensor, token_indices, emb, P):
    tok = token_indices.astype(jnp.int32).reshape(NW * CHUNKS, IDX_CHUNK)
    x = _sc_gather(tok, emb)
    d2 = jnp.asarray(_D_CONST) * P.reshape(1, Q)
    return _dense_tc(k_tensor.reshape(-1, 1), x, d2)


# speedup vs baseline: 4.0071x; 3.8581x over previous
import functools

import numpy as np
import jax
import jax.numpy as jnp
from jax import lax
from jax.experimental import pallas as pl
from jax.experimental.pallas import tpu as pltpu
from jax.experimental.pallas import tpu_sc as plsc

M = 64
Q = M * M
B_TOTAL = 16384
BB = 512

NC = 2
NS = 16
NW = NC * NS
ROWS_PER_W = B_TOTAL // NW
IDX_CHUNK = 128
CHUNKS = ROWS_PER_W // IDX_CHUNK

_W_CONST = (1.0 / (np.arange(Q, dtype=np.float64) + 2.0)).astype(
    np.float32).reshape(1, Q)
_D_CONST = np.tile(np.eye(M, dtype=np.float32), (1, M))
_E_CONST = np.repeat(np.eye(M, dtype=np.float32), M, axis=0)

_S1 = np.float32(2.0 * np.pi)
_S3 = np.float32(-(2.0 * np.pi) ** 3 / 6.0)
_S5 = np.float32((2.0 * np.pi) ** 5 / 120.0)
_S7 = np.float32(-(2.0 * np.pi) ** 7 / 5040.0)
_S9 = np.float32((2.0 * np.pi) ** 9 / 362880.0)


def _sc_gather(tok2d, emb):
    mesh = plsc.VectorSubcoreMesh(core_axis_name="c", subcore_axis_name="s")

    @functools.partial(
        pl.kernel,
        mesh=mesh,
        out_type=jax.ShapeDtypeStruct((B_TOTAL, M), jnp.float32),
        scratch_types=[
            pltpu.VMEM((CHUNKS, IDX_CHUNK), jnp.int32),
            pltpu.VMEM((ROWS_PER_W, M), jnp.float32),
            pltpu.SemaphoreType.DMA,
        ],
        compiler_params=pltpu.CompilerParams(use_tc_tiling_on_sc=False),
    )
    def gk(tok_hbm, emb_hbm, out_hbm, idx_v, rows_v, sem):
        wid = lax.axis_index("s") * NC + lax.axis_index("c")
        pltpu.sync_copy(tok_hbm.at[pl.ds(wid * CHUNKS, CHUNKS)], idx_v)
        copies = [
            pltpu.async_copy(
                emb_hbm.at[idx_v.at[j]],
                rows_v.at[pl.ds(j * IDX_CHUNK, IDX_CHUNK)],
                sem,
            )
            for j in range(CHUNKS)
        ]
        for c in copies:
            c.wait()
        pltpu.sync_copy(rows_v, out_hbm.at[pl.ds(wid * ROWS_PER_W, ROWS_PER_W)])

    return gk(tok2d, emb)


def _dd_body(k_ref, x_ref, d2_ref, e_ref, w_ref, o_ref):
    f = k_ref[...] * w_ref[...]
    r = f - jnp.floor(f)
    w = jnp.abs(r - 0.5) - 0.25
    w2 = w * w
    g = w * (_S1 + w2 * (_S3 + w2 * (_S5 + w2 * (_S7 + w2 * _S9))))
    xt = jnp.dot(x_ref[...].astype(jnp.bfloat16), d2_ref[...],
                 preferred_element_type=jnp.float32)
    o_ref[...] = jnp.dot((g * xt).astype(jnp.bfloat16), e_ref[...],
                         preferred_element_type=jnp.float32)


def _dense_tc(k2d, x, d2):
    grid = (k2d.shape[0] // BB,)
    return pl.pallas_call(
        _dd_body,
        grid=grid,
        in_specs=[
            pl.BlockSpec((BB, 1), lambda i: (i, 0)),
            pl.BlockSpec((BB, M), lambda i: (i, 0)),
            pl.BlockSpec((M, Q), lambda i: (0, 0)),
            pl.BlockSpec((Q, M), lambda i: (0, 0)),
            pl.BlockSpec((1, Q), lambda i: (0, 0)),
        ],
        out_specs=pl.BlockSpec((BB, M), lambda i: (i, 0)),
        out_shape=jax.ShapeDtypeStruct((k2d.shape[0], M), jnp.float32),
        compiler_params=pltpu.CompilerParams(
            dimension_semantics=("parallel",)),
    )(k2d, x, d2, jnp.asarray(_E_CONST).astype(jnp.bfloat16),
      jnp.asarray(_W_CONST))


def kernel(k_tensor, token_indices, emb, P):
    tok = token_indices.astype(jnp.int32).reshape(NW * CHUNKS, IDX_CHUNK)
    x = _sc_gather(tok, emb)
    d2 = (jnp.asarray(_D_CONST) * P.reshape(1, Q)).astype(jnp.bfloat16)
    return _dense_tc(k_tensor.reshape(-1, 1), x, d2)


# speedup vs baseline: 4.4483x; 1.1101x over previous
import functools

import numpy as np
import jax
import jax.numpy as jnp
from jax import lax
from jax.experimental import pallas as pl
from jax.experimental.pallas import tpu as pltpu
from jax.experimental.pallas import tpu_sc as plsc

M = 64
Q = M * M
B_TOTAL = 16384
BB = 512

NC = 2
NS = 16
NW = NC * NS
ROWS_PER_W = B_TOTAL // NW
IDX_CHUNK = 128
CHUNKS = ROWS_PER_W // IDX_CHUNK

_W_CONST = (1.0 / (np.arange(Q, dtype=np.float64) + 2.0)).astype(
    np.float32).reshape(1, Q)
_D_CONST = np.tile(np.eye(M, dtype=np.float32), (1, M))
_E_CONST = np.repeat(np.eye(M, dtype=np.float32), M, axis=0)

_S1 = np.float32(-(2.0 * np.pi))
_S3 = np.float32((2.0 * np.pi) ** 3 / 6.0)
_S5 = np.float32(-(2.0 * np.pi) ** 5 / 120.0)
_S7 = np.float32((2.0 * np.pi) ** 7 / 5040.0)


def _sc_gather(tok2d, emb):
    mesh = plsc.VectorSubcoreMesh(core_axis_name="c", subcore_axis_name="s")

    @functools.partial(
        pl.kernel,
        mesh=mesh,
        out_type=jax.ShapeDtypeStruct((B_TOTAL, M), jnp.float32),
        scratch_types=[
            pltpu.VMEM((CHUNKS, IDX_CHUNK), jnp.int32),
            pltpu.VMEM((ROWS_PER_W, M), jnp.float32),
            pltpu.SemaphoreType.DMA,
        ],
        compiler_params=pltpu.CompilerParams(use_tc_tiling_on_sc=False),
    )
    def gk(tok_hbm, emb_hbm, out_hbm, idx_v, rows_v, sem):
        wid = lax.axis_index("s") * NC + lax.axis_index("c")
        pltpu.sync_copy(tok_hbm.at[pl.ds(wid * CHUNKS, CHUNKS)], idx_v)
        copies = [
            pltpu.async_copy(
                emb_hbm.at[idx_v.at[j]],
                rows_v.at[pl.ds(j * IDX_CHUNK, IDX_CHUNK)],
                sem,
            )
            for j in range(CHUNKS)
        ]
        for c in copies:
            c.wait()
        pltpu.sync_copy(rows_v, out_hbm.at[pl.ds(wid * ROWS_PER_W, ROWS_PER_W)])

    return gk(tok2d, emb)


def _dd_body(k_ref, x_ref, d2_ref, e_ref, w_ref, o_ref):
    f = k_ref[...] * w_ref[...]
    w = jnp.abs(f - lax.round(f, lax.RoundingMethod.TO_NEAREST_EVEN)) - 0.25
    w2 = w * w
    g = w * (_S1 + w2 * (_S3 + w2 * (_S5 + w2 * _S7)))
    xt = jnp.dot(x_ref[...].astype(jnp.bfloat16), d2_ref[...],
                 preferred_element_type=jnp.float32)
    o_ref[...] = jnp.dot((g * xt).astype(jnp.bfloat16), e_ref[...],
                         preferred_element_type=jnp.float32)


def _dense_tc(k2d, x, d2):
    grid = (k2d.shape[0] // BB,)
    return pl.pallas_call(
        _dd_body,
        grid=grid,
        in_specs=[
            pl.BlockSpec((BB, 1), lambda i: (i, 0)),
            pl.BlockSpec((BB, M), lambda i: (i, 0)),
            pl.BlockSpec((M, Q), lambda i: (0, 0)),
            pl.BlockSpec((Q, M), lambda i: (0, 0)),
            pl.BlockSpec((1, Q), lambda i: (0, 0)),
        ],
        out_specs=pl.BlockSpec((BB, M), lambda i: (i, 0)),
        out_shape=jax.ShapeDtypeStruct((k2d.shape[0], M), jnp.float32),
        compiler_params=pltpu.CompilerParams(
            dimension_semantics=("parallel",)),
    )(k2d, x, d2, jnp.asarray(_E_CONST).astype(jnp.bfloat16),
      jnp.asarray(_W_CONST))


def kernel(k_tensor, token_indices, emb, P):
    tok = token_indices.astype(jnp.int32).reshape(NW * CHUNKS, IDX_CHUNK)
    x = _sc_gather(tok, emb)
    d2 = (jnp.asarray(_D_CONST) * P.reshape(1, Q)).astype(jnp.bfloat16)
    return _dense_tc(k_tensor.reshape(-1, 1), x, d2)


# speedup vs baseline: 4.5385x; 1.0203x over previous
import functools

import numpy as np
import jax
import jax.numpy as jnp
from jax import lax
from jax.experimental import pallas as pl
from jax.experimental.pallas import tpu as pltpu
from jax.experimental.pallas import tpu_sc as plsc

M = 64
Q = M * M
B_TOTAL = 16384
BB = 1024

NC = 2
NS = 16
NW = NC * NS
ROWS_PER_W = B_TOTAL // NW
IDX_CHUNK = 128
CHUNKS = ROWS_PER_W // IDX_CHUNK

_W_CONST = (1.0 / (np.arange(Q, dtype=np.float64) + 2.0)).astype(
    np.float32).reshape(1, Q)
_D_CONST = np.tile(np.eye(M, dtype=np.float32), (1, M))
_E_CONST = np.repeat(np.eye(M, dtype=np.float32), M, axis=0)

_S1 = np.float32(-(2.0 * np.pi))
_S3 = np.float32((2.0 * np.pi) ** 3 / 6.0)
_S5 = np.float32(-(2.0 * np.pi) ** 5 / 120.0)
_S7 = np.float32((2.0 * np.pi) ** 7 / 5040.0)


def _sc_gather(tok1d, emb):
    mesh = plsc.VectorSubcoreMesh(core_axis_name="c", subcore_axis_name="s")

    @functools.partial(
        pl.kernel,
        mesh=mesh,
        out_type=jax.ShapeDtypeStruct((B_TOTAL, M), jnp.float32),
        scratch_types=[
            pltpu.VMEM((ROWS_PER_W,), jnp.int32),
            pltpu.VMEM((ROWS_PER_W, M), jnp.float32),
            pltpu.SemaphoreType.DMA,
        ],
        compiler_params=pltpu.CompilerParams(use_tc_tiling_on_sc=False),
    )
    def gk(tok_hbm, emb_hbm, out_hbm, idx_v, rows_v, sem):
        wid = lax.axis_index("s") * NC + lax.axis_index("c")
        pltpu.sync_copy(tok_hbm.at[pl.ds(wid * ROWS_PER_W, ROWS_PER_W)], idx_v)
        copies = [
            pltpu.async_copy(
                emb_hbm.at[idx_v.at[pl.ds(j * IDX_CHUNK, IDX_CHUNK)]],
                rows_v.at[pl.ds(j * IDX_CHUNK, IDX_CHUNK)],
                sem,
            )
            for j in range(CHUNKS)
        ]
        for c in copies:
            c.wait()
        pltpu.sync_copy(rows_v, out_hbm.at[pl.ds(wid * ROWS_PER_W, ROWS_PER_W)])

    return gk(tok1d, emb)


def _dd_body(k_ref, x_ref, d2_ref, e_ref, w_ref, o_ref):
    f = k_ref[...] * w_ref[...]
    w = jnp.abs(f - lax.round(f, lax.RoundingMethod.TO_NEAREST_EVEN)) - 0.25
    w2 = w * w
    g = w * (_S1 + w2 * (_S3 + w2 * (_S5 + w2 * _S7)))
    xt = jnp.dot(x_ref[...].astype(jnp.bfloat16), d2_ref[...],
                 preferred_element_type=jnp.float32)
    o_ref[...] = jnp.dot((g * xt).astype(jnp.bfloat16), e_ref[...],
                         preferred_element_type=jnp.float32)


def _dense_tc(k2d, x, d2):
    grid = (k2d.shape[0] // BB,)
    return pl.pallas_call(
        _dd_body,
        grid=grid,
        in_specs=[
            pl.BlockSpec((BB, 1), lambda i: (i, 0)),
            pl.BlockSpec((BB, M), lambda i: (i, 0)),
            pl.BlockSpec((M, Q), lambda i: (0, 0)),
            pl.BlockSpec((Q, M), lambda i: (0, 0)),
            pl.BlockSpec((1, Q), lambda i: (0, 0)),
        ],
        out_specs=pl.BlockSpec((BB, M), lambda i: (i, 0)),
        out_shape=jax.ShapeDtypeStruct((k2d.shape[0], M), jnp.float32),
        compiler_params=pltpu.CompilerParams(
            dimension_semantics=("parallel",)),
    )(k2d, x, d2, jnp.asarray(_E_CONST).astype(jnp.bfloat16),
      jnp.asarray(_W_CONST))


def kernel(k_tensor, token_indices, emb, P):
    x = _sc_gather(token_indices.astype(jnp.int32), emb)
    d2 = (jnp.asarray(_D_CONST) * P.reshape(1, Q)).astype(jnp.bfloat16)
    return _dense_tc(k_tensor.reshape(-1, 1), x, d2)


# speedup vs baseline: 4.9394x; 1.0883x over previous
import functools

import numpy as np
import jax
import jax.numpy as jnp
from jax import lax
from jax.experimental import pallas as pl
from jax.experimental.pallas import tpu as pltpu
from jax.experimental.pallas import tpu_sc as plsc

M = 64
Q = M * M
B_TOTAL = 16384
BB = 1024

NC = 2
NS = 16
NW = NC * NS
ROWS_PER_W = B_TOTAL // NW
IDX_CHUNK = 128
CHUNKS = ROWS_PER_W // IDX_CHUNK

_W_CONST = (1.0 / (np.arange(Q, dtype=np.float64) + 2.0)).astype(
    np.float32).reshape(1, Q)
_D_CONST = np.tile(np.eye(M, dtype=np.float32), (1, M))
_E_CONST = np.repeat(np.eye(M, dtype=np.float32), M, axis=0)

_S1 = np.float32(-6.28128131)
_S3 = np.float32(41.09534543)
_S5 = np.float32(-73.5871216)


def _sc_gather(tok1d, emb):
    mesh = plsc.VectorSubcoreMesh(core_axis_name="c", subcore_axis_name="s")

    @functools.partial(
        pl.kernel,
        mesh=mesh,
        out_type=jax.ShapeDtypeStruct((B_TOTAL, M), jnp.float32),
        scratch_types=[
            pltpu.VMEM((ROWS_PER_W,), jnp.int32),
            pltpu.VMEM((ROWS_PER_W, M), jnp.float32),
            pltpu.SemaphoreType.DMA,
        ],
        compiler_params=pltpu.CompilerParams(use_tc_tiling_on_sc=False),
    )
    def gk(tok_hbm, emb_hbm, out_hbm, idx_v, rows_v, sem):
        wid = lax.axis_index("s") * NC + lax.axis_index("c")
        pltpu.sync_copy(tok_hbm.at[pl.ds(wid * ROWS_PER_W, ROWS_PER_W)], idx_v)
        copies = [
            pltpu.async_copy(
                emb_hbm.at[idx_v.at[pl.ds(j * IDX_CHUNK, IDX_CHUNK)]],
                rows_v.at[pl.ds(j * IDX_CHUNK, IDX_CHUNK)],
                sem,
            )
            for j in range(CHUNKS)
        ]
        for c in copies:
            c.wait()
        pltpu.sync_copy(rows_v, out_hbm.at[pl.ds(wid * ROWS_PER_W, ROWS_PER_W)])

    return gk(tok1d, emb)


def _dd_body(k_ref, x_ref, d2_ref, e_ref, w_ref, o_ref):
    f = k_ref[...] * w_ref[...]
    w = jnp.abs(f - lax.round(f, lax.RoundingMethod.TO_NEAREST_EVEN)) - 0.25
    w2 = w * w
    g = w * (_S1 + w2 * (_S3 + w2 * _S5))
    xt = jnp.dot(x_ref[...].astype(jnp.bfloat16), d2_ref[...],
                 preferred_element_type=jnp.float32)
    o_ref[...] = jnp.dot((g * xt).astype(jnp.bfloat16), e_ref[...],
                         preferred_element_type=jnp.float32)


def _dense_tc(k2d, x, d2):
    grid = (k2d.shape[0] // BB,)
    return pl.pallas_call(
        _dd_body,
        grid=grid,
        in_specs=[
            pl.BlockSpec((BB, 1), lambda i: (i, 0)),
            pl.BlockSpec((BB, M), lambda i: (i, 0)),
            pl.BlockSpec((M, Q), lambda i: (0, 0)),
            pl.BlockSpec((Q, M), lambda i: (0, 0)),
            pl.BlockSpec((1, Q), lambda i: (0, 0)),
        ],
        out_specs=pl.BlockSpec((BB, M), lambda i: (i, 0)),
        out_shape=jax.ShapeDtypeStruct((k2d.shape[0], M), jnp.float32),
        compiler_params=pltpu.CompilerParams(
            dimension_semantics=("parallel",)),
    )(k2d, x, d2, jnp.asarray(_E_CONST).astype(jnp.bfloat16),
      jnp.asarray(_W_CONST))


def kernel(k_tensor, token_indices, emb, P):
    x = _sc_gather(token_indices.astype(jnp.int32), emb)
    d2 = (jnp.asarray(_D_CONST) * P.reshape(1, Q)).astype(jnp.bfloat16)
    return _dense_tc(k_tensor.reshape(-1, 1), x, d2)
